# baseline (speedup 1.0000x reference)
import functools

import jax
import jax.numpy as jnp
from jax.experimental import pallas as pl
from jax.experimental.pallas import tpu as pltpu

B = 128
HIDDEN = 1024
NUM_EXPERTS = 64
TOP_K = 8
INTER = 512
NORM_MIN = 1e-12
E_PER = 4


def _moe_kernel(x_ref, rw_ref, bias_ref, gate_ref, up_ref, down_ref,
                out_ref, logits_ref, comb_ref, xb_ref):
    i = pl.program_id(0)

    @pl.when(i == 0)
    def _router():
        x = x_ref[...]
        logits = jnp.dot(x, rw_ref[...].T, preferred_element_type=jnp.float32)
        logits_ref[...] = logits
        probs = jax.nn.softmax(logits, axis=-1)
        scores = probs + bias_ref[...]
        col = jax.lax.broadcasted_iota(jnp.int32, scores.shape, 1)
        work = scores
        comb = jnp.zeros_like(probs)
        for _ in range(TOP_K):
            m = jnp.max(work, axis=-1, keepdims=True)
            first = jnp.min(jnp.where(work == m, col, NUM_EXPERTS),
                            axis=-1, keepdims=True)
            sel = col == first
            comb = comb + jnp.where(sel, probs, 0.0)
            work = jnp.where(sel, -jnp.inf, work)
        denom = jnp.maximum(jnp.sum(comb, axis=-1, keepdims=True), NORM_MIN)
        comb_ref[...] = comb / denom
        out_ref[...] = jnp.zeros_like(out_ref)
        xb_ref[...] = x.astype(jnp.bfloat16)

    xb = xb_ref[...]
    ecol = jax.lax.broadcasted_iota(jnp.int32, (B, NUM_EXPERTS), 1)
    comb = comb_ref[...]
    for j in range(E_PER):
        g = jnp.dot(xb, gate_ref[j].astype(jnp.bfloat16),
                    preferred_element_type=jnp.float32)
        u = jnp.dot(xb, up_ref[j].astype(jnp.bfloat16),
                    preferred_element_type=jnp.float32)
        w = jnp.sum(jnp.where(ecol == i * E_PER + j, comb, 0.0),
                    axis=-1, keepdims=True)
        hw = (jax.nn.silu(g) * u * w).astype(jnp.bfloat16)
        out_ref[...] += jnp.dot(hw, down_ref[j].astype(jnp.bfloat16),
                                preferred_element_type=jnp.float32)


@functools.partial(jax.jit, static_argnames=("interpret",))
def kernel(hidden_states, router_weight, e_bias, gate_w, up_w, down_w,
           interpret=False):
    shape = hidden_states.shape
    x = hidden_states.reshape(-1, HIDDEN)
    out, logits = pl.pallas_call(
        _moe_kernel,
        grid=(NUM_EXPERTS // E_PER,),
        in_specs=[
            pl.BlockSpec((B, HIDDEN), lambda i: (0, 0)),
            pl.BlockSpec((NUM_EXPERTS, HIDDEN), lambda i: (0, 0)),
            pl.BlockSpec((1, NUM_EXPERTS), lambda i: (0, 0)),
            pl.BlockSpec((E_PER, HIDDEN, INTER), lambda i: (i, 0, 0)),
            pl.BlockSpec((E_PER, HIDDEN, INTER), lambda i: (i, 0, 0)),
            pl.BlockSpec((E_PER, INTER, HIDDEN), lambda i: (i, 0, 0)),
        ],
        out_specs=[
            pl.BlockSpec((B, HIDDEN), lambda i: (0, 0)),
            pl.BlockSpec((B, NUM_EXPERTS), lambda i: (0, 0)),
        ],
        out_shape=[
            jax.ShapeDtypeStruct((B, HIDDEN), jnp.float32),
            jax.ShapeDtypeStruct((B, NUM_EXPERTS), jnp.float32),
        ],
        scratch_shapes=[pltpu.VMEM((B, NUM_EXPERTS), jnp.float32),
                        pltpu.VMEM((B, HIDDEN), jnp.bfloat16)],
        interpret=interpret,
    )(x, router_weight, e_bias, gate_w, up_w, down_w)
    return out.reshape(shape), logits


# speedup vs baseline: 1.0291x; 1.0291x over previous
import functools

import jax
import jax.numpy as jnp
from jax.experimental import pallas as pl
from jax.experimental.pallas import tpu as pltpu

B = 128
HIDDEN = 1024
NUM_EXPERTS = 64
TOP_K = 8
INTER = 512
NORM_MIN = 1e-12
E_PER = 2


def _moe_kernel(x_ref, rw_ref, bias_ref, gate_ref, up_ref, down_ref,
                out_ref, logits_ref, comb_ref, xb_ref):
    i = pl.program_id(0)

    @pl.when(i == 0)
    def _router():
        x = x_ref[...]
        logits = jnp.dot(x, rw_ref[...].T, preferred_element_type=jnp.float32)
        logits_ref[...] = logits
        probs = jax.nn.softmax(logits, axis=-1)
        scores = probs + bias_ref[...]
        col = jax.lax.broadcasted_iota(jnp.int32, scores.shape, 1)
        work = scores
        comb = jnp.zeros_like(probs)
        for _ in range(TOP_K):
            m = jnp.max(work, axis=-1, keepdims=True)
            first = jnp.min(jnp.where(work == m, col, NUM_EXPERTS),
                            axis=-1, keepdims=True)
            sel = col == first
            comb = comb + jnp.where(sel, probs, 0.0)
            work = jnp.where(sel, -jnp.inf, work)
        denom = jnp.maximum(jnp.sum(comb, axis=-1, keepdims=True), NORM_MIN)
        comb_ref[...] = comb / denom
        out_ref[...] = jnp.zeros_like(out_ref)
        xb_ref[...] = x.astype(jnp.bfloat16)

    xb = xb_ref[...]
    ecol = jax.lax.broadcasted_iota(jnp.int32, (B, NUM_EXPERTS), 1)
    comb = comb_ref[...]
    for j in range(E_PER):
        g = jnp.dot(xb, gate_ref[j].astype(jnp.bfloat16),
                    preferred_element_type=jnp.float32)
        u = jnp.dot(xb, up_ref[j].astype(jnp.bfloat16),
                    preferred_element_type=jnp.float32)
        w = jnp.sum(jnp.where(ecol == i * E_PER + j, comb, 0.0),
                    axis=-1, keepdims=True)
        hw = (jax.nn.silu(g) * u * w).astype(jnp.bfloat16)
        out_ref[...] += jnp.dot(hw, down_ref[j].astype(jnp.bfloat16),
                                preferred_element_type=jnp.float32)


@functools.partial(jax.jit, static_argnames=("interpret",))
def kernel(hidden_states, router_weight, e_bias, gate_w, up_w, down_w,
           interpret=False):
    shape = hidden_states.shape
    x = hidden_states.reshape(-1, HIDDEN)
    out, logits = pl.pallas_call(
        _moe_kernel,
        grid=(NUM_EXPERTS // E_PER,),
        in_specs=[
            pl.BlockSpec((B, HIDDEN), lambda i: (0, 0)),
            pl.BlockSpec((NUM_EXPERTS, HIDDEN), lambda i: (0, 0)),
            pl.BlockSpec((1, NUM_EXPERTS), lambda i: (0, 0)),
            pl.BlockSpec((E_PER, HIDDEN, INTER), lambda i: (i, 0, 0)),
            pl.BlockSpec((E_PER, HIDDEN, INTER), lambda i: (i, 0, 0)),
            pl.BlockSpec((E_PER, INTER, HIDDEN), lambda i: (i, 0, 0)),
        ],
        out_specs=[
            pl.BlockSpec((B, HIDDEN), lambda i: (0, 0)),
            pl.BlockSpec((B, NUM_EXPERTS), lambda i: (0, 0)),
        ],
        out_shape=[
            jax.ShapeDtypeStruct((B, HIDDEN), jnp.float32),
            jax.ShapeDtypeStruct((B, NUM_EXPERTS), jnp.float32),
        ],
        scratch_shapes=[pltpu.VMEM((B, NUM_EXPERTS), jnp.float32),
                        pltpu.VMEM((B, HIDDEN), jnp.bfloat16)],
        interpret=interpret,
    )(x, router_weight, e_bias, gate_w, up_w, down_w)
    return out.reshape(shape), logits
